# Initial kernel scaffold; baseline (speedup 1.0000x reference)
import functools

import jax
import jax.numpy as jnp
from jax import lax
from jax.experimental import pallas as pl
from jax.experimental.pallas import tpu as pltpu
from jax.experimental.pallas import tpu_sc as plsc

T = 9216
D = 64
V = 8192

TB = 1024
CB = 2048
N_TB = T // TB
N_CB = V // CB

NC, NS = 2, 16
NW = NC * NS
BPW = T // NW
CH = 96
NCH = BPW // CH


def _argmin_body(x_ref, e_ref, idx_ref):
    x = x_ref[...]
    run_min = jnp.full((TB,), jnp.inf, dtype=jnp.float32)
    run_idx = jnp.zeros((TB,), dtype=jnp.int32)
    for c in range(N_CB):
        ec = e_ref[pl.ds(c * CB, CB), :]
        e2 = jnp.sum(ec * ec, axis=1)
        d = -2.0 * jax.lax.dot_general(
            x, ec, (((1,), (1,)), ((), ())),
            preferred_element_type=jnp.float32) + e2[None, :]
        m = jnp.min(d, axis=1)
        col = jax.lax.broadcasted_iota(jnp.int32, d.shape, 1) + c * CB
        masked = jnp.where(d == m[:, None], col, jnp.int32(2**30))
        i = jnp.min(masked, axis=1)
        upd = m < run_min
        run_idx = jnp.where(upd, i, run_idx)
        run_min = jnp.where(upd, m, run_min)
    idx_ref[...] = run_idx.reshape(TB // 128, 128)


def _nearest_idx(xf, E):
    return pl.pallas_call(
        _argmin_body,
        grid=(N_TB,),
        in_specs=[
            pl.BlockSpec((TB, D), lambda i: (i, 0)),
            pl.BlockSpec((V, D), lambda i: (0, 0)),
        ],
        out_specs=pl.BlockSpec((TB // 128, 128), lambda i: (i, 0)),
        out_shape=jax.ShapeDtypeStruct((T // 128, 128), jnp.int32),
    )(xf, E)


def _sc_gather(E, idx2d):
    mesh = plsc.VectorSubcoreMesh(core_axis_name="c", subcore_axis_name="s")

    @functools.partial(
        pl.kernel, mesh=mesh,
        out_type=jax.ShapeDtypeStruct((T, D), jnp.float32),
        scratch_types=[
            pltpu.VMEM((NCH, CH), jnp.int32),
            pltpu.VMEM((BPW, D), jnp.float32),
            pltpu.SemaphoreType.DMA,
        ],
    )
    def gather_k(table_hbm, idx_hbm, out_hbm, idx_v, rows_v, sem):
        wid = lax.axis_index("s") * NC + lax.axis_index("c")
        pltpu.sync_copy(idx_hbm.at[pl.ds(wid * NCH, NCH)], idx_v)
        copies = []
        for j in range(NCH):
            copies.append(pltpu.async_copy(
                table_hbm.at[idx_v.at[j]],
                rows_v.at[pl.ds(j * CH, CH)], sem))
        for cp in copies:
            cp.wait()
        pltpu.sync_copy(rows_v, out_hbm.at[pl.ds(wid * BPW, BPW)])

    return gather_k(E, idx2d)


def kernel(x, E):
    batch_dim = x.shape[:-1]
    xf = x.reshape(-1, D)
    idx = _nearest_idx(xf, E)
    idx2d = idx.reshape(NW * NCH, CH)
    values = _sc_gather(E, idx2d)
    return values.reshape(*batch_dim, D)


# speedup vs baseline: 1.2517x; 1.2517x over previous
import functools

import jax
import jax.numpy as jnp
from jax import lax
from jax.experimental import pallas as pl
from jax.experimental.pallas import tpu as pltpu
from jax.experimental.pallas import tpu_sc as plsc

T = 9216
D = 64
V = 8192

TB = 1024
CB = 2048
N_TB = T // TB
N_CB = V // CB

NC, NS = 2, 16
NW = NC * NS
BPW = T // NW
CH = 96
NCH = BPW // CH


def _argmin_body(x_ref, e_ref, idx_ref):
    x = x_ref[...]
    run_min = jnp.full((TB,), jnp.inf, dtype=jnp.float32)
    run_idx = jnp.zeros((TB,), dtype=jnp.int32)
    for c in range(N_CB):
        ec = e_ref[pl.ds(c * CB, CB), :]
        e2 = jnp.sum(ec * ec, axis=1)
        d = -2.0 * jax.lax.dot_general(
            x, ec, (((1,), (1,)), ((), ())),
            preferred_element_type=jnp.float32) + e2[None, :]
        m = jnp.min(d, axis=1)
        col = jax.lax.broadcasted_iota(jnp.int32, d.shape, 1) + c * CB
        masked = jnp.where(d == m[:, None], col, jnp.int32(2**30))
        i = jnp.min(masked, axis=1)
        upd = m < run_min
        run_idx = jnp.where(upd, i, run_idx)
        run_min = jnp.where(upd, m, run_min)
    idx_ref[...] = run_idx.reshape(TB // 128, 128)


def _nearest_idx(xf, E):
    return pl.pallas_call(
        _argmin_body,
        grid=(N_TB,),
        in_specs=[
            pl.BlockSpec((TB, D), lambda i: (i, 0)),
            pl.BlockSpec((V, D), lambda i: (0, 0)),
        ],
        out_specs=pl.BlockSpec((TB // 128, 128), lambda i: (i, 0)),
        out_shape=jax.ShapeDtypeStruct((T // 128, 128), jnp.int32),
    )(xf, E)


def _sc_gather(E, idx3d):
    mesh = plsc.VectorSubcoreMesh(core_axis_name="c", subcore_axis_name="s")

    @functools.partial(
        pl.kernel, mesh=mesh,
        compiler_params=pltpu.CompilerParams(use_tc_tiling_on_sc=False),
        out_type=jax.ShapeDtypeStruct((T, D), jnp.float32),
        scratch_types=[
            pltpu.VMEM((NCH, CH), jnp.int32),
            pltpu.VMEM((BPW, D), jnp.float32),
            pltpu.SemaphoreType.DMA,
        ],
    )
    def gather_k(table_hbm, idx_hbm, out_hbm, idx_v, rows_v, sem):
        wid = lax.axis_index("s") * NC + lax.axis_index("c")
        pltpu.sync_copy(idx_hbm.at[wid], idx_v)
        copies = []
        for j in range(NCH):
            copies.append(pltpu.async_copy(
                table_hbm.at[idx_v.at[j]],
                rows_v.at[pl.ds(j * CH, CH)], sem))
        for cp in copies:
            cp.wait()
        pltpu.sync_copy(rows_v, out_hbm.at[pl.ds(wid * BPW, BPW)])

    return gather_k(E, idx3d)


def kernel(x, E):
    batch_dim = x.shape[:-1]
    xf = x.reshape(-1, D)
    idx = _nearest_idx(xf, E)
    idx3d = idx.reshape(NW, NCH, CH)
    values = _sc_gather(E, idx3d)
    return values.reshape(*batch_dim, D)


# speedup vs baseline: 1.6471x; 1.3159x over previous
import functools

import jax
import jax.numpy as jnp
from jax import lax
from jax.experimental import pallas as pl
from jax.experimental.pallas import tpu as pltpu
from jax.experimental.pallas import tpu_sc as plsc

T = 9216
D = 64
V = 8192

TB = 1024
CB = 2048
N_TB = T // TB
N_CB = V // CB

NC, NS = 2, 16
NW = NC * NS
BPW = T // NW
CH = 96
NCH = BPW // CH


def _argmin_body(x_ref, e_ref, idx_ref, e2_ref):
    @pl.when(pl.program_id(0) == 0)
    def _():
        ef = e_ref[...]
        e2 = jnp.sum(ef * ef, axis=1)
        e2_ref[...] = e2.reshape(V // 128, 128)

    x = x_ref[...]
    best_val = jnp.full((TB, 128), jnp.inf, dtype=jnp.float32)
    best_gid = jnp.zeros((TB, 128), dtype=jnp.float32)
    for c in range(N_CB):
        ec = e_ref[pl.ds(c * CB, CB), :]
        raw = jax.lax.dot_general(
            x, ec, (((1,), (1,)), ((), ())),
            preferred_element_type=jnp.float32)
        for g in range(CB // 128):
            G = c * (CB // 128) + g
            dg = -2.0 * raw[:, g * 128:(g + 1) * 128] + e2_ref[G][None, :]
            lt = dg < best_val
            best_gid = jnp.where(lt, jnp.float32(G), best_gid)
            best_val = jnp.minimum(dg, best_val)
    m = jnp.min(best_val, axis=1)
    lane = jax.lax.broadcasted_iota(jnp.int32, (TB, 128), 1).astype(jnp.float32)
    cand = jnp.where(best_val == m[:, None],
                     best_gid * 128.0 + lane, jnp.float32(1e9))
    idx_ref[...] = jnp.min(cand, axis=1).astype(jnp.int32).reshape(TB // 128, 128)


def _nearest_idx(xf, E):
    return pl.pallas_call(
        _argmin_body,
        grid=(N_TB,),
        in_specs=[
            pl.BlockSpec((TB, D), lambda i: (i, 0)),
            pl.BlockSpec((V, D), lambda i: (0, 0)),
        ],
        out_specs=pl.BlockSpec((TB // 128, 128), lambda i: (i, 0)),
        out_shape=jax.ShapeDtypeStruct((T // 128, 128), jnp.int32),
        scratch_shapes=[pltpu.VMEM((V // 128, 128), jnp.float32)],
    )(xf, E)


def _sc_gather(E, idx3d):
    mesh = plsc.VectorSubcoreMesh(core_axis_name="c", subcore_axis_name="s")

    @functools.partial(
        pl.kernel, mesh=mesh,
        compiler_params=pltpu.CompilerParams(use_tc_tiling_on_sc=False),
        out_type=jax.ShapeDtypeStruct((T, D), jnp.float32),
        scratch_types=[
            pltpu.VMEM((NCH, CH), jnp.int32),
            pltpu.VMEM((BPW, D), jnp.float32),
            pltpu.SemaphoreType.DMA,
        ],
    )
    def gather_k(table_hbm, idx_hbm, out_hbm, idx_v, rows_v, sem):
        wid = lax.axis_index("s") * NC + lax.axis_index("c")
        pltpu.sync_copy(idx_hbm.at[wid], idx_v)
        copies = []
        for j in range(NCH):
            copies.append(pltpu.async_copy(
                table_hbm.at[idx_v.at[j]],
                rows_v.at[pl.ds(j * CH, CH)], sem))
        for cp in copies:
            cp.wait()
        pltpu.sync_copy(rows_v, out_hbm.at[pl.ds(wid * BPW, BPW)])

    return gather_k(E, idx3d)


def kernel(x, E):
    batch_dim = x.shape[:-1]
    xf = x.reshape(-1, D)
    idx = _nearest_idx(xf, E)
    idx3d = idx.reshape(NW, NCH, CH)
    values = _sc_gather(E, idx3d)
    return values.reshape(*batch_dim, D)
